# baseline (speedup 1.0000x reference)
import functools

import jax
import jax.numpy as jnp
from jax import lax
from jax.experimental import pallas as pl
from jax.experimental.pallas import tpu as pltpu
from jax.experimental.pallas import tpu_sc as plsc

N = 10000
E = 320000
D = 128
NP = 10240
NC = 2
NS = 16
EB = E // NS
SCHK = 800
CH = 80
ROWB = 1024
EBLK = 2560


def _pad8(col0, col1):
    zeros = jnp.zeros((D, 6), jnp.float32)
    return jnp.concatenate([col0, col1, zeros], axis=1)


def _tc_pre_body(ea_ref, fr0_ref, fr1_ref, a0_ref, a1_ref, s3_ref):
    r0 = jnp.dot(fr0_ref[...], a0_ref[2 * D:3 * D, :],
                 preferred_element_type=jnp.float32)
    r1 = jnp.dot(fr1_ref[...], a1_ref[2 * D:3 * D, :],
                 preferred_element_type=jnp.float32)
    R = _pad8(r0, r1)
    s3_ref[...] = lax.dot_general(R, ea_ref[...], (((0,), (1,)), ((), ())),
                                  preferred_element_type=jnp.float32)


def _tc_pre(edge_attr, fc_r_w0, fc_r_w1, attn_w0, attn_w1):
    wfull = pl.BlockSpec((D, D), lambda b: (0, 0))
    afull = pl.BlockSpec((3 * D, 1), lambda b: (0, 0))
    return pl.pallas_call(
        _tc_pre_body,
        grid=(E // EBLK,),
        in_specs=[pl.BlockSpec((EBLK, D), lambda b: (b, 0)),
                  wfull, wfull, afull, afull],
        out_specs=pl.BlockSpec((8, EBLK), lambda b: (0, b)),
        out_shape=jax.ShapeDtypeStruct((8, E), jnp.float32),
    )(edge_attr, fc_r_w0, fc_r_w1, attn_w0, attn_w1)


def _project(z, loopw_ref, aw_ref, z_ref, zl_ref, s12_ref):
    z_ref[...] = z
    zl_ref[...] = jnp.dot(z, loopw_ref[...], preferred_element_type=jnp.float32)
    W = _pad8(aw_ref[0:D, :], aw_ref[D:2 * D, :])
    s12_ref[...] = lax.dot_general(W, z, (((0,), (1,)), ((), ())),
                                   preferred_element_type=jnp.float32)


def _combine(a1_ref, a2_ref, den_ref, frw_ref, zlp_ref):
    den = den_ref[...]
    dens = jnp.where(den > 0.0, den, 1.0)
    msg = (a1_ref[...]
           + jnp.dot(a2_ref[...], frw_ref[...],
                     preferred_element_type=jnp.float32)) / dens
    return jnp.maximum(msg + zlp_ref[...], 0.0)


def _tc0_body(x_ref, fcw_ref, loopw_ref, aw_ref, z_ref, zl_ref, s12_ref):
    z = jnp.dot(x_ref[...], fcw_ref[...], preferred_element_type=jnp.float32)
    _project(z, loopw_ref, aw_ref, z_ref, zl_ref, s12_ref)


def _tc1_body(a1_ref, a2_ref, den_ref, frw_ref, zlp_ref, fcw_ref, loopw_ref,
              aw_ref, z_ref, zl_ref, s12_ref):
    h = _combine(a1_ref, a2_ref, den_ref, frw_ref, zlp_ref)
    z = jnp.dot(h, fcw_ref[...], preferred_element_type=jnp.float32)
    _project(z, loopw_ref, aw_ref, z_ref, zl_ref, s12_ref)


def _tc2_body(a1_ref, a2_ref, den_ref, frw_ref, zlp_ref, out_ref):
    out_ref[...] = _combine(a1_ref, a2_ref, den_ref, frw_ref, zlp_ref)


_ROWBS = pl.BlockSpec((ROWB, D), lambda b: (b, 0))
_A1BS = pl.BlockSpec((ROWB, D), lambda b: (b, 0))
_A2BS = pl.BlockSpec((ROWB, D), lambda b: (NP // ROWB + b, 0))
_DENBS = pl.BlockSpec((ROWB, 1), lambda b: (b, 0))
_WBS = pl.BlockSpec((D, D), lambda b: (0, 0))
_ABS = pl.BlockSpec((3 * D, 1), lambda b: (0, 0))
_PROJ_OUT = dict(
    out_specs=[pl.BlockSpec((ROWB, D), lambda b: (b, 0)),
               pl.BlockSpec((ROWB, D), lambda b: (b, 0)),
               pl.BlockSpec((8, ROWB), lambda b: (0, b))],
    out_shape=[jax.ShapeDtypeStruct((NP, D), jnp.float32),
               jax.ShapeDtypeStruct((NP, D), jnp.float32),
               jax.ShapeDtypeStruct((8, NP), jnp.float32)],
)


def _tc0(x, fc_w, loop_w, attn_w):
    return pl.pallas_call(
        _tc0_body, grid=(NP // ROWB,),
        in_specs=[_ROWBS, _WBS, _WBS, _ABS], **_PROJ_OUT,
    )(x, fc_w, loop_w, attn_w)


def _tc1(agg, den, fc_r_w, zl_prev, fc_w, loop_w, attn_w):
    return pl.pallas_call(
        _tc1_body, grid=(NP // ROWB,),
        in_specs=[_A1BS, _A2BS, _DENBS, _WBS, _ROWBS, _WBS, _WBS, _ABS],
        **_PROJ_OUT,
    )(agg, agg, den, fc_r_w, zl_prev, fc_w, loop_w, attn_w)


def _tc2(agg, den, fc_r_w, zl_prev):
    return pl.pallas_call(
        _tc2_body, grid=(NP // ROWB,),
        in_specs=[_A1BS, _A2BS, _DENBS, _WBS, _ROWBS],
        out_specs=pl.BlockSpec((ROWB, D), lambda b: (b, 0)),
        out_shape=jax.ShapeDtypeStruct((NP, D), jnp.float32),
    )(agg, agg, den, fc_r_w, zl_prev)


_MESH = plsc.VectorSubcoreMesh(core_axis_name="c", subcore_axis_name="s",
                               num_cores=NC, num_subcores=NS)


@functools.partial(
    pl.kernel,
    out_type=jax.ShapeDtypeStruct((2 * NP + 80, D), jnp.float32),
    mesh=_MESH,
    compiler_params=pltpu.CompilerParams(needs_layout_passes=False),
    scratch_types=[
        pltpu.VMEM((NP,), jnp.float32),
        pltpu.VMEM((NP,), jnp.float32),
        pltpu.VMEM((3, SCHK), jnp.int32),
        pltpu.VMEM((SCHK,), jnp.float32),
        pltpu.VMEM((CH, D), jnp.float32),
        pltpu.VMEM((CH, D), jnp.float32),
        pltpu.VMEM((CH,), jnp.int32),
        pltpu.VMEM((CH,), jnp.int32),
        pltpu.VMEM((CH,), jnp.int32),
        pltpu.VMEM((CH,), jnp.int32),
        pltpu.VMEM((1024,), jnp.float32),
        pltpu.VMEM((8, D), jnp.float32),
        pltpu.VMEM_SHARED((NP, D), jnp.float32),
        pltpu.VMEM_SHARED((NP,), jnp.float32),
        pltpu.SemaphoreType.DMA,
        pltpu.SemaphoreType.DMA,
        pltpu.SemaphoreType.DMA,
        pltpu.SemaphoreType.DMA,
        pltpu.SemaphoreType.DMA,
        pltpu.SemaphoreType.DMA,
    ],
)
def _sc_layer(epk_h, s1_h, s2_h, z_h, ea_h, out_h,
              s1v, s2v, esv, exv, zb0, zb1,
              idxg0, idxg1, idxd0, idxd1, dtmp, dtmp2, aggsh, dshared,
              semg0, semg1, sems0, sems1, seme0, seme1):
    c = lax.axis_index("c")
    s = lax.axis_index("s")
    base = s * EB
    zb = (zb0, zb1)
    idxg = (idxg0, idxg1)
    idxd = (idxd0, idxd1)
    semg = (semg0, semg1)
    sems = (sems0, sems1)
    seme = (seme0, seme1)
    CQ = SCHK // CH

    pltpu.sync_copy(s1_h, s1v)
    pltpu.sync_copy(s2_h, s2v)

    @pl.loop(0, CH)
    def _zrow(r):
        for q in range(D // 16):
            zb0[r, pl.ds(q * 16, 16)] = jnp.zeros((16,), jnp.float32)

    @pl.loop(0, 1024 // 16)
    def _zden(i):
        dtmp[pl.ds(i * 16, 16)] = jnp.zeros((16,), jnp.float32)

    for k in range(640 // CH):
        pltpu.sync_copy(zb0, aggsh.at[pl.ds(s * 640 + k * CH, CH)])

    @pl.when(c == 0)
    def _():
        pltpu.sync_copy(dtmp.at[pl.ds(0, 640)], dshared.at[pl.ds(s * 640, 640)])

    plsc.subcore_barrier()

    @pl.loop(0, EB // SCHK)
    def _super(u):
        sbase = base + u * SCHK
        pltpu.sync_copy(epk_h.at[s * (EB // SCHK) + u], esv)

        def _stage_gather(p, co):
            @pl.when(c == 0)
            def _():
                for k in range(CH // 16):
                    idxg[p][pl.ds(k * 16, 16)] = esv[0, pl.ds(co + k * 16, 16)]
                pltpu.async_copy(z_h.at[idxg[p]], zb[p], semg[p])

            @pl.when(c == 1)
            def _():
                pltpu.async_copy(ea_h.at[pl.ds(sbase + co, CH)], zb[p],
                                 semg[p])

        def _wait_gather(p):
            pltpu.make_async_copy(ea_h.at[pl.ds(sbase, CH)], zb[p],
                                  semg[p]).wait()

        def _scale(p, co):
            @pl.loop(0, CH // 16, unroll=5)
            def _sc(k):
                o = co + k * 16
                sv = esv[0, pl.ds(o, 16)]
                dv = esv[1, pl.ds(o, 16)]
                s3 = plsc.bitcast(esv[2, pl.ds(o, 16)], jnp.float32)
                g2 = plsc.load_gather(s2v, [dv])
                a = plsc.load_gather(s1v, [sv]) + g2 + s3
                e = jnp.maximum(a, 0.01 * a)
                cc = jnp.maximum(g2, 0.01 * g2)
                exl = jnp.exp(e - cc)
                exv[pl.ds(o, 16)] = exl
                for i in range(16):
                    w = exl[i]
                    e2 = k * 16 + i
                    for q in range(D // 16):
                        zb[p][e2, pl.ds(q * 16, 16)] = (
                            zb[p][e2, pl.ds(q * 16, 16)] * w)

        def _start_scatter(p, co):
            for k in range(CH // 16):
                idxd[p][pl.ds(k * 16, 16)] = esv[1, pl.ds(co + k * 16, 16)]
            pltpu.async_copy(zb[p], aggsh.at[idxd[p]], sems[p], add=True)

            @pl.when(c == 0)
            def _():
                pltpu.async_copy(exv.at[pl.ds(co, CH)], dshared.at[idxd[p]],
                                 seme[p], add=True)

        def _wait_scatter(p):
            pltpu.make_async_copy(zb[p], aggsh.at[idxd[p]], sems[p]).wait()

            @pl.when(c == 0)
            def _():
                pltpu.make_async_copy(exv.at[pl.ds(0, CH)],
                                      dshared.at[idxd[p]], seme[p]).wait()

        _stage_gather(0, 0)

        @pl.loop(0, CQ // 2)
        def _pair(t):
            coa = 2 * t * CH
            cob = coa + CH
            _wait_gather(0)

            @pl.when(t > 0)
            def _():
                _wait_scatter(1)

            _stage_gather(1, cob)
            _scale(0, coa)
            _start_scatter(0, coa)
            _wait_gather(1)

            @pl.when(t < CQ // 2 - 1)
            def _():
                _wait_scatter(0)
                _stage_gather(0, coa + 2 * CH)

            _scale(1, cob)
            _start_scatter(1, cob)

        _wait_scatter(0)
        _wait_scatter(1)

    plsc.subcore_barrier()

    @pl.when(c == 0)
    def _():
        pltpu.sync_copy(aggsh.at[pl.ds(s * 640, 640)],
                        out_h.at[pl.ds(s * 640, 640)])

    @pl.when(c == 1)
    def _():
        pltpu.sync_copy(aggsh.at[pl.ds(s * 640, 640)],
                        out_h.at[pl.ds(NP + s * 640, 640)])

    @pl.when(jnp.logical_and(c == 0, s < 10))
    def _():
        pltpu.sync_copy(dshared.at[pl.ds(s * 1024, 1024)], dtmp)

        @pl.loop(0, 8)
        def _d2(r):
            for q in range(D // 16):
                dtmp2[r, pl.ds(q * 16, 16)] = dtmp[pl.ds(r * 128 + q * 16, 16)]

        pltpu.sync_copy(dtmp2, out_h.at[pl.ds(2 * NP + s * 8, 8)])


def _pack_edges(src, dst, s3row):
    nblk = E // SCHK
    s3b = jax.lax.bitcast_convert_type(s3row, jnp.int32)
    return jnp.stack([jnp.reshape(src, (nblk, SCHK)),
                      jnp.reshape(dst, (nblk, SCHK)),
                      jnp.reshape(s3b, (nblk, SCHK))], axis=1)


def _layer_parts(epk, s1, s2, z, ea):
    agg = _sc_layer(epk, s1, s2, z, ea)
    return agg, jnp.reshape(agg[2 * NP:], (NP, 1))


def kernel(x, edge_index, edge_attr, fc_w0, fc_r_w0, attn_w0, loop_w0,
           fc_w1, fc_r_w1, attn_w1, loop_w1):
    src = edge_index[0]
    dst = edge_index[1]
    xp = jnp.pad(x, ((0, NP - N), (0, 0)))

    s3T = _tc_pre(edge_attr, fc_r_w0, fc_r_w1, attn_w0, attn_w1)
    epk0 = _pack_edges(src, dst, s3T[0])
    epk1 = _pack_edges(src, dst, s3T[1])

    z0, zl0, s12 = _tc0(xp, fc_w0, loop_w0, attn_w0)
    agg0, den0 = _layer_parts(epk0, s12[0], s12[1], z0, edge_attr)

    z1, zl1, s12b = _tc1(agg0, den0, fc_r_w0, zl0, fc_w1, loop_w1, attn_w1)
    agg1, den1 = _layer_parts(epk1, s12b[0], s12b[1], z1, edge_attr)

    return _tc2(agg1, den1, fc_r_w1, zl1)[:N]


# speedup vs baseline: 1.3339x; 1.3339x over previous
import functools

import jax
import jax.numpy as jnp
from jax import lax
from jax.experimental import pallas as pl
from jax.experimental.pallas import tpu as pltpu
from jax.experimental.pallas import tpu_sc as plsc

N = 10000
E = 320000
D = 128
NP = 10240
NC = 2
NS = 16
EB = E // NS
SCHK = 800
CH = 80
ROWB = 1024
EBLK = 2560


def _pad8(col0, col1):
    zeros = jnp.zeros((D, 6), jnp.float32)
    return jnp.concatenate([col0, col1, zeros], axis=1)


def _tc_pre_body(ea_ref, fr0_ref, fr1_ref, a0_ref, a1_ref, s3_ref):
    r0 = jnp.dot(fr0_ref[...], a0_ref[2 * D:3 * D, :],
                 preferred_element_type=jnp.float32)
    r1 = jnp.dot(fr1_ref[...], a1_ref[2 * D:3 * D, :],
                 preferred_element_type=jnp.float32)
    R = _pad8(r0, r1)
    s3_ref[...] = lax.dot_general(R, ea_ref[...], (((0,), (1,)), ((), ())),
                                  preferred_element_type=jnp.float32)


def _tc_pre(edge_attr, fc_r_w0, fc_r_w1, attn_w0, attn_w1):
    wfull = pl.BlockSpec((D, D), lambda b: (0, 0))
    afull = pl.BlockSpec((3 * D, 1), lambda b: (0, 0))
    return pl.pallas_call(
        _tc_pre_body,
        grid=(E // EBLK,),
        in_specs=[pl.BlockSpec((EBLK, D), lambda b: (b, 0)),
                  wfull, wfull, afull, afull],
        out_specs=pl.BlockSpec((8, EBLK), lambda b: (0, b)),
        out_shape=jax.ShapeDtypeStruct((8, E), jnp.float32),
    )(edge_attr, fc_r_w0, fc_r_w1, attn_w0, attn_w1)


def _project(z, loopw_ref, aw_ref, z_ref, zl_ref, s12_ref):
    z_ref[...] = z
    zl_ref[...] = jnp.dot(z, loopw_ref[...], preferred_element_type=jnp.float32)
    W = _pad8(aw_ref[0:D, :], aw_ref[D:2 * D, :])
    s12_ref[...] = lax.dot_general(W, z, (((0,), (1,)), ((), ())),
                                   preferred_element_type=jnp.float32)


def _combine(a1_ref, a2_ref, den_ref, frw_ref, zlp_ref):
    den = den_ref[...]
    dens = jnp.where(den > 0.0, den, 1.0)
    msg = (a1_ref[...]
           + jnp.dot(a2_ref[...], frw_ref[...],
                     preferred_element_type=jnp.float32)) / dens
    return jnp.maximum(msg + zlp_ref[...], 0.0)


def _tc0_body(x_ref, fcw_ref, loopw_ref, aw_ref, z_ref, zl_ref, s12_ref):
    z = jnp.dot(x_ref[...], fcw_ref[...], preferred_element_type=jnp.float32)
    _project(z, loopw_ref, aw_ref, z_ref, zl_ref, s12_ref)


def _tc1_body(a1_ref, a2_ref, den_ref, frw_ref, zlp_ref, fcw_ref, loopw_ref,
              aw_ref, z_ref, zl_ref, s12_ref):
    h = _combine(a1_ref, a2_ref, den_ref, frw_ref, zlp_ref)
    z = jnp.dot(h, fcw_ref[...], preferred_element_type=jnp.float32)
    _project(z, loopw_ref, aw_ref, z_ref, zl_ref, s12_ref)


def _tc2_body(a1_ref, a2_ref, den_ref, frw_ref, zlp_ref, out_ref):
    out_ref[...] = _combine(a1_ref, a2_ref, den_ref, frw_ref, zlp_ref)


_ROWBS = pl.BlockSpec((ROWB, D), lambda b: (b, 0))
_A1BS = pl.BlockSpec((ROWB, D), lambda b: (b, 0))
_A2BS = pl.BlockSpec((ROWB, D), lambda b: (NP // ROWB + b, 0))
_DENBS = pl.BlockSpec((ROWB, 1), lambda b: (b, 0))
_WBS = pl.BlockSpec((D, D), lambda b: (0, 0))
_ABS = pl.BlockSpec((3 * D, 1), lambda b: (0, 0))
_PROJ_OUT = dict(
    out_specs=[pl.BlockSpec((ROWB, D), lambda b: (b, 0)),
               pl.BlockSpec((ROWB, D), lambda b: (b, 0)),
               pl.BlockSpec((8, ROWB), lambda b: (0, b))],
    out_shape=[jax.ShapeDtypeStruct((NP, D), jnp.float32),
               jax.ShapeDtypeStruct((NP, D), jnp.float32),
               jax.ShapeDtypeStruct((8, NP), jnp.float32)],
)


def _tc0(x, fc_w, loop_w, attn_w):
    return pl.pallas_call(
        _tc0_body, grid=(NP // ROWB,),
        in_specs=[_ROWBS, _WBS, _WBS, _ABS], **_PROJ_OUT,
    )(x, fc_w, loop_w, attn_w)


def _tc1(agg, den, fc_r_w, zl_prev, fc_w, loop_w, attn_w):
    return pl.pallas_call(
        _tc1_body, grid=(NP // ROWB,),
        in_specs=[_A1BS, _A2BS, _DENBS, _WBS, _ROWBS, _WBS, _WBS, _ABS],
        **_PROJ_OUT,
    )(agg, agg, den, fc_r_w, zl_prev, fc_w, loop_w, attn_w)


def _tc2(agg, den, fc_r_w, zl_prev):
    return pl.pallas_call(
        _tc2_body, grid=(NP // ROWB,),
        in_specs=[_A1BS, _A2BS, _DENBS, _WBS, _ROWBS],
        out_specs=pl.BlockSpec((ROWB, D), lambda b: (b, 0)),
        out_shape=jax.ShapeDtypeStruct((NP, D), jnp.float32),
    )(agg, agg, den, fc_r_w, zl_prev)


_MESH = plsc.VectorSubcoreMesh(core_axis_name="c", subcore_axis_name="s",
                               num_cores=NC, num_subcores=NS)


@functools.partial(
    pl.kernel,
    out_type=jax.ShapeDtypeStruct((2 * NP + 80, D), jnp.float32),
    mesh=_MESH,
    compiler_params=pltpu.CompilerParams(needs_layout_passes=False),
    scratch_types=[
        pltpu.VMEM((NP,), jnp.float32),
        pltpu.VMEM((NP,), jnp.float32),
        pltpu.VMEM((3, SCHK), jnp.int32),
        pltpu.VMEM((SCHK,), jnp.float32),
        pltpu.VMEM((CH, D), jnp.float32),
        pltpu.VMEM((CH, D), jnp.float32),
        pltpu.VMEM((CH,), jnp.int32),
        pltpu.VMEM((CH,), jnp.int32),
        pltpu.VMEM((CH,), jnp.int32),
        pltpu.VMEM((CH,), jnp.int32),
        pltpu.VMEM((1024,), jnp.float32),
        pltpu.VMEM((8, D), jnp.float32),
        pltpu.VMEM_SHARED((NP, D), jnp.float32),
        pltpu.VMEM_SHARED((NP,), jnp.float32),
        pltpu.SemaphoreType.DMA,
        pltpu.SemaphoreType.DMA,
        pltpu.SemaphoreType.DMA,
        pltpu.SemaphoreType.DMA,
        pltpu.SemaphoreType.DMA,
        pltpu.SemaphoreType.DMA,
    ],
)
def _sc_layer(epk_h, s1_h, s2_h, z_h, ea_h, out_h,
              s1v, s2v, esv, exv, zb0, zb1,
              idxg0, idxg1, idxd0, idxd1, dtmp, dtmp2, aggsh, dshared,
              semg0, semg1, sems0, sems1, seme0, seme1):
    c = lax.axis_index("c")
    s = lax.axis_index("s")
    base = s * EB
    zb = (zb0, zb1)
    idxg = (idxg0, idxg1)
    idxd = (idxd0, idxd1)
    semg = (semg0, semg1)
    sems = (sems0, sems1)
    seme = (seme0, seme1)
    CQ = SCHK // CH

    pltpu.sync_copy(s1_h, s1v)
    pltpu.sync_copy(s2_h, s2v)

    @pl.loop(0, CH)
    def _zrow(r):
        for q in range(D // 16):
            zb0[r, pl.ds(q * 16, 16)] = jnp.zeros((16,), jnp.float32)

    @pl.loop(0, 1024 // 16)
    def _zden(i):
        dtmp[pl.ds(i * 16, 16)] = jnp.zeros((16,), jnp.float32)

    for k in range(640 // CH):
        pltpu.sync_copy(zb0, aggsh.at[pl.ds(s * 640 + k * CH, CH)])

    @pl.when(c == 0)
    def _():
        pltpu.sync_copy(dtmp.at[pl.ds(0, 640)], dshared.at[pl.ds(s * 640, 640)])

    plsc.subcore_barrier()

    @pl.loop(0, EB // SCHK)
    def _super(u):
        sbase = base + u * SCHK
        pltpu.sync_copy(epk_h.at[s * (EB // SCHK) + u], esv)

        def _stage_gather(p, co):
            @pl.when(c == 0)
            def _():
                for k in range(CH // 16):
                    idxg[p][pl.ds(k * 16, 16)] = esv[0, pl.ds(co + k * 16, 16)]
                pltpu.async_copy(z_h.at[idxg[p]], zb[p], semg[p])

            @pl.when(c == 1)
            def _():
                pltpu.async_copy(ea_h.at[pl.ds(sbase + co, CH)], zb[p],
                                 semg[p])

        def _wait_gather(p):
            pltpu.make_async_copy(ea_h.at[pl.ds(sbase, CH)], zb[p],
                                  semg[p]).wait()

        def _scale(p, co):
            @pl.loop(0, CH // 16)
            def _sc(k):
                o = co + k * 16
                sv = esv[0, pl.ds(o, 16)]
                dv = esv[1, pl.ds(o, 16)]
                s3 = plsc.bitcast(esv[2, pl.ds(o, 16)], jnp.float32)
                g2 = plsc.load_gather(s2v, [dv])
                a = plsc.load_gather(s1v, [sv]) + g2 + s3
                e = jnp.maximum(a, 0.01 * a)
                cc = jnp.maximum(g2, 0.01 * g2)
                exl = jnp.exp(e - cc)
                exv[pl.ds(o, 16)] = exl
                for i in range(16):
                    w = exl[i]
                    e2 = k * 16 + i
                    for q in range(D // 16):
                        zb[p][e2, pl.ds(q * 16, 16)] = (
                            zb[p][e2, pl.ds(q * 16, 16)] * w)

        def _start_scatter(p, co):
            for k in range(CH // 16):
                idxd[p][pl.ds(k * 16, 16)] = esv[1, pl.ds(co + k * 16, 16)]
            pltpu.async_copy(zb[p], aggsh.at[idxd[p]], sems[p], add=True)

            @pl.when(c == 0)
            def _():
                pltpu.async_copy(exv.at[pl.ds(co, CH)], dshared.at[idxd[p]],
                                 seme[p], add=True)

        def _wait_scatter(p):
            pltpu.make_async_copy(zb[p], aggsh.at[idxd[p]], sems[p]).wait()

            @pl.when(c == 0)
            def _():
                pltpu.make_async_copy(exv.at[pl.ds(0, CH)],
                                      dshared.at[idxd[p]], seme[p]).wait()

        _stage_gather(0, 0)

        @pl.loop(0, CQ // 2)
        def _pair(t):
            coa = 2 * t * CH
            cob = coa + CH
            _wait_gather(0)

            @pl.when(t > 0)
            def _():
                _wait_scatter(1)

            _stage_gather(1, cob)
            _scale(0, coa)
            _start_scatter(0, coa)
            _wait_gather(1)

            @pl.when(t < CQ // 2 - 1)
            def _():
                _wait_scatter(0)
                _stage_gather(0, coa + 2 * CH)

            _scale(1, cob)
            _start_scatter(1, cob)

        _wait_scatter(0)
        _wait_scatter(1)

    plsc.subcore_barrier()

    @pl.when(c == 0)
    def _():
        pltpu.sync_copy(aggsh.at[pl.ds(s * 640, 640)],
                        out_h.at[pl.ds(s * 640, 640)])

    @pl.when(c == 1)
    def _():
        pltpu.sync_copy(aggsh.at[pl.ds(s * 640, 640)],
                        out_h.at[pl.ds(NP + s * 640, 640)])

    @pl.when(jnp.logical_and(c == 0, s < 10))
    def _():
        pltpu.sync_copy(dshared.at[pl.ds(s * 1024, 1024)], dtmp)

        @pl.loop(0, 8)
        def _d2(r):
            for q in range(D // 16):
                dtmp2[r, pl.ds(q * 16, 16)] = dtmp[pl.ds(r * 128 + q * 16, 16)]

        pltpu.sync_copy(dtmp2, out_h.at[pl.ds(2 * NP + s * 8, 8)])


def _pack_edges(src, dst, s3row):
    nblk = E // SCHK
    s3b = jax.lax.bitcast_convert_type(s3row, jnp.int32)
    return jnp.stack([jnp.reshape(src, (nblk, SCHK)),
                      jnp.reshape(dst, (nblk, SCHK)),
                      jnp.reshape(s3b, (nblk, SCHK))], axis=1)


def _layer_parts(epk, s1, s2, z, ea):
    agg = _sc_layer(epk, s1, s2, z, ea)
    return agg, jnp.reshape(agg[2 * NP:], (NP, 1))


def kernel(x, edge_index, edge_attr, fc_w0, fc_r_w0, attn_w0, loop_w0,
           fc_w1, fc_r_w1, attn_w1, loop_w1):
    src = edge_index[0]
    dst = edge_index[1]
    xp = jnp.pad(x, ((0, NP - N), (0, 0)))

    s3T = _tc_pre(edge_attr, fc_r_w0, fc_r_w1, attn_w0, attn_w1)
    epk0 = _pack_edges(src, dst, s3T[0])
    epk1 = _pack_edges(src, dst, s3T[1])

    z0, zl0, s12 = _tc0(xp, fc_w0, loop_w0, attn_w0)
    agg0, den0 = _layer_parts(epk0, s12[0], s12[1], z0, edge_attr)

    z1, zl1, s12b = _tc1(agg0, den0, fc_r_w0, zl0, fc_w1, loop_w1, attn_w1)
    agg1, den1 = _layer_parts(epk1, s12b[0], s12b[1], z1, edge_attr)

    return _tc2(agg1, den1, fc_r_w1, zl1)[:N]
